# baseline (speedup 1.0000x reference)
import functools

import jax
import jax.numpy as jnp
from jax import lax
from jax.experimental import pallas as pl
from jax.experimental.pallas import tpu as pltpu
from jax.experimental.pallas import tpu_sc as plsc

BATCH = 16384
EMB = 64
NUM_CORES = 2
NUM_SUBCORES = 16
NW = NUM_CORES * NUM_SUBCORES
BPW = BATCH // NW
CHUNK = 128
NCHUNK = BPW // CHUNK


def _sc_gather(user_idx, movie_idx, user_emb, movie_emb):
    mesh = plsc.VectorSubcoreMesh(core_axis_name="c", subcore_axis_name="s")

    @functools.partial(
        pl.kernel,
        mesh=mesh,
        compiler_params=pltpu.CompilerParams(use_tc_tiling_on_sc=False),
        out_type=(
            jax.ShapeDtypeStruct((BATCH, EMB), jnp.float32),
            jax.ShapeDtypeStruct((BATCH, EMB), jnp.float32),
        ),
        scratch_types=[
            pltpu.VMEM((BPW,), jnp.int32),
            pltpu.VMEM((BPW,), jnp.int32),
            pltpu.VMEM((BPW, EMB), jnp.float32),
            pltpu.VMEM((BPW, EMB), jnp.float32),
            pltpu.SemaphoreType.DMA,
        ],
    )
    def gather_kernel(uidx_hbm, midx_hbm, uemb_hbm, memb_hbm,
                      out_u, out_m,
                      uidx_v, midx_v, urows_v, mrows_v, sem):
        wid = lax.axis_index("s") * NUM_CORES + lax.axis_index("c")
        base = wid * BPW
        pltpu.sync_copy(uidx_hbm.at[pl.ds(base, BPW)], uidx_v)
        pltpu.sync_copy(midx_hbm.at[pl.ds(base, BPW)], midx_v)
        copies = []
        for j in range(NCHUNK):
            sl = pl.ds(j * CHUNK, CHUNK)
            copies.append(
                pltpu.async_copy(uemb_hbm.at[uidx_v.at[sl]], urows_v.at[sl], sem))
            copies.append(
                pltpu.async_copy(memb_hbm.at[midx_v.at[sl]], mrows_v.at[sl], sem))
        for c in copies:
            c.wait()
        pltpu.sync_copy(urows_v, out_u.at[pl.ds(base, BPW)])
        pltpu.sync_copy(mrows_v, out_m.at[pl.ds(base, BPW)])

    return gather_kernel(user_idx, movie_idx, user_emb, movie_emb)


def _tc_mlp(u, m, w1u, w1m, b1, w2, b2, w3, b3):
    BB = 2048
    grid = (BATCH // BB,)

    def body(u_ref, m_ref, w1u_ref, w1m_ref, b1_ref, w2_ref, b2_ref,
             w3_ref, b3_ref, o_ref):
        h = jnp.dot(u_ref[...], w1u_ref[...], preferred_element_type=jnp.float32)
        h = h + jnp.dot(m_ref[...], w1m_ref[...], preferred_element_type=jnp.float32)
        h = jnp.maximum(h + b1_ref[...], 0.0)
        h2 = jnp.dot(h, w2_ref[...], preferred_element_type=jnp.float32)
        h2 = jnp.maximum(h2 + b2_ref[...], 0.0)
        o_ref[...] = jnp.sum(h2 * w3_ref[...], axis=1, keepdims=True) + b3_ref[...]

    rep = lambda shape: pl.BlockSpec(shape, lambda i: (0, 0))
    return pl.pallas_call(
        body,
        grid=grid,
        in_specs=[
            pl.BlockSpec((BB, EMB), lambda i: (i, 0)),
            pl.BlockSpec((BB, EMB), lambda i: (i, 0)),
            rep((EMB, 128)),
            rep((EMB, 128)),
            rep((1, 128)),
            rep((128, 64)),
            rep((1, 64)),
            rep((1, 64)),
            rep((1, 1)),
        ],
        out_specs=pl.BlockSpec((BB, 1), lambda i: (i, 0)),
        out_shape=jax.ShapeDtypeStruct((BATCH, 1), jnp.float32),
    )(u, m, w1u, w1m, b1, w2, b2, w3, b3)


def kernel(user_idx, movie_idx, user_emb, movie_emb, W1, b1, W2, b2, W3, b3):
    u, m = _sc_gather(user_idx.astype(jnp.int32), movie_idx.astype(jnp.int32),
                      user_emb, movie_emb)
    w1u = W1[:, :EMB].T
    w1m = W1[:, EMB:].T
    w2 = W2.T
    return _tc_mlp(u, m, w1u, w1m, b1.reshape(1, 128), w2, b2.reshape(1, 64),
                   W3.reshape(1, 64), b3.reshape(1, 1))
